# baseline (speedup 1.0000x reference)
import jax
import jax.numpy as jnp
from jax.experimental import pallas as pl

_RADII = [0.2, 0.4]
_RATIO = 0.25
_K = 64


def _fps(pos, m):
    n = pos.shape[0]
    idxs = jnp.zeros((m,), dtype=jnp.int32)
    dists = jnp.full((n,), jnp.inf, dtype=jnp.float32)

    def body(i, state):
        dists, idxs = state
        last = idxs[i - 1]
        d = jnp.sum((pos - pos[last]) ** 2, axis=1)
        dists = jnp.minimum(dists, d)
        nxt = jnp.argmax(dists).astype(jnp.int32)
        idxs = idxs.at[i].set(nxt)
        return (dists, idxs)

    dists, idxs = jax.lax.fori_loop(1, m, body, (dists, idxs))
    return idxs


def kernel(x, pos, batch, w10, b10, w11, b11, w20, b20, w21, b21):
    n = pos.shape[0]
    m = int(n * _RATIO)
    idx = _fps(pos, m)
    pos_q = pos[idx]

    d2 = jnp.sum((pos_q[:, None, :] - pos[None, :, :]) ** 2, axis=-1)
    negd, cols = jax.lax.top_k(-d2, _K)
    d2sel = -negd

    outs = []
    for r, (Wa, ba, Wb, bb) in zip(_RADII,
                                   [(w10, b10, w11, b11), (w20, b20, w21, b21)]):
        a = x @ Wa[:64] + pos @ Wa[64:] + ba
        v = pos_q @ Wa[64:]
        g = a[cols.reshape(-1)].reshape(m, _K, -1)
        h1 = g - v[:, None, :]
        vmask = d2sel <= r * r
        mf = vmask.astype(jnp.float32)[:, :, None]
        cnt = jnp.sum(mf)
        mu = jnp.sum(h1 * mf, axis=(0, 1)) / cnt
        var = jnp.sum(((h1 - mu) ** 2) * mf, axis=(0, 1)) / cnt
        hn = jax.nn.relu((h1 - mu) / jnp.sqrt(var + 1e-5))
        h2 = hn @ Wb + bb
        h2m = jnp.where(vmask[:, :, None], h2, -jnp.inf)
        outs.append(jnp.max(h2m, axis=1))
    return jnp.concatenate(outs, axis=1), pos_q, batch[idx]


# speedup vs baseline: 3.1044x; 3.1044x over previous
import functools

import jax
import jax.numpy as jnp
from jax.experimental import pallas as pl
from jax.experimental.pallas import tpu as pltpu

_RADII = [0.2, 0.4]
_RATIO = 0.25
_K = 64

_N = 10000
_M = 2500
_NL = 1264
_NPAD = 8 * _NL
_ML = 320


def _fps_body(posp_ref, idx_ref, posq_ref, dists_ref):
    px = posp_ref[0]
    py = posp_ref[1]
    pz = posp_ref[2]
    iota2d = (jax.lax.broadcasted_iota(jnp.int32, (8, _NL), 0) * _NL
              + jax.lax.broadcasted_iota(jnp.int32, (8, _NL), 1))
    valid = iota2d < _N
    dists_ref[...] = jnp.where(valid, jnp.inf, -jnp.inf).astype(jnp.float32)
    siota = (jax.lax.broadcasted_iota(jnp.int32, (8, _ML), 0) * _ML
             + jax.lax.broadcasted_iota(jnp.int32, (8, _ML), 1))

    c0x, c0y, c0z = posp_ref[0, 0, 0], posp_ref[1, 0, 0], posp_ref[2, 0, 0]
    m0 = siota == 0
    idx_ref[...] = jnp.where(m0, 0, 0).astype(jnp.int32)
    posq_ref[0] = jnp.where(m0, c0x, 0.0)
    posq_ref[1] = jnp.where(m0, c0y, 0.0)
    posq_ref[2] = jnp.where(m0, c0z, 0.0)

    def body(i, c):
        cx, cy, cz = c
        dx = px - cx
        dy = py - cy
        dz = pz - cz
        d = dx * dx + dy * dy + dz * dz
        dn = jnp.minimum(dists_ref[...], d)
        dists_ref[...] = dn
        mx = jnp.max(dn)
        nxt = jnp.min(jnp.where(dn == mx, iota2d, jnp.int32(2 ** 30)))
        hit = iota2d == nxt
        ninf = jnp.float32(-jnp.inf)
        nx = jnp.max(jnp.where(hit, px, ninf))
        ny = jnp.max(jnp.where(hit, py, ninf))
        nz = jnp.max(jnp.where(hit, pz, ninf))
        ms = siota == i
        idx_ref[...] = jnp.where(ms, nxt, idx_ref[...])
        posq_ref[0] = jnp.where(ms, nx, posq_ref[0])
        posq_ref[1] = jnp.where(ms, ny, posq_ref[1])
        posq_ref[2] = jnp.where(ms, nz, posq_ref[2])
        return (nx, ny, nz)

    jax.lax.fori_loop(1, _M, body, (c0x, c0y, c0z))


@functools.partial(jax.jit)
def _fps_pallas(pos):
    posT = jnp.transpose(pos)
    posp = jnp.pad(posT, ((0, 0), (0, _NPAD - _N))).reshape(3, 8, _NL)
    idx8, posq8 = pl.pallas_call(
        _fps_body,
        out_shape=[
            jax.ShapeDtypeStruct((8, _ML), jnp.int32),
            jax.ShapeDtypeStruct((3, 8, _ML), jnp.float32),
        ],
        scratch_shapes=[pltpu.VMEM((8, _NL), jnp.float32)],
    )(posp)
    idx = idx8.reshape(-1)[:_M]
    pos_q = posq8.reshape(3, -1)[:, :_M].T
    return idx, pos_q


def kernel(x, pos, batch, w10, b10, w11, b11, w20, b20, w21, b21):
    n = pos.shape[0]
    m = _M
    idx, pos_q = _fps_pallas(pos)

    d2 = jnp.sum((pos_q[:, None, :] - pos[None, :, :]) ** 2, axis=-1)
    negd, cols = jax.lax.top_k(-d2, _K)
    d2sel = -negd

    outs = []
    for r, (Wa, ba, Wb, bb) in zip(_RADII,
                                   [(w10, b10, w11, b11), (w20, b20, w21, b21)]):
        a = x @ Wa[:64] + pos @ Wa[64:] + ba
        v = pos_q @ Wa[64:]
        g = a[cols.reshape(-1)].reshape(m, _K, -1)
        h1 = g - v[:, None, :]
        vmask = d2sel <= r * r
        mf = vmask.astype(jnp.float32)[:, :, None]
        cnt = jnp.sum(mf)
        mu = jnp.sum(h1 * mf, axis=(0, 1)) / cnt
        var = jnp.sum(((h1 - mu) ** 2) * mf, axis=(0, 1)) / cnt
        hn = jax.nn.relu((h1 - mu) / jnp.sqrt(var + 1e-5))
        h2 = hn @ Wb + bb
        h2m = jnp.where(vmask[:, :, None], h2, -jnp.inf)
        outs.append(jnp.max(h2m, axis=1))
    return jnp.concatenate(outs, axis=1), pos_q, batch[idx]
